# baseline (speedup 1.0000x reference)
import functools

import jax
import jax.numpy as jnp
from jax import lax
from jax.experimental import pallas as pl
from jax.experimental.pallas import tpu as pltpu
from jax.experimental.pallas import tpu_sc as plsc

NC = 2
NS = 16
NW = NC * NS
LANES = 16


def _relayout_pack(table_t, cols):
    d, v = table_t.shape
    half = cols // 2
    nblk = pl.cdiv(v, cols)

    def body(tl, tr, tout):
        tout[:, 0:d] = tl[...].T
        tout[:, d:2 * d] = tr[...].T

    return pl.pallas_call(
        body,
        grid=(nblk,),
        in_specs=[
            pl.BlockSpec((d, half), lambda g: (0, g)),
            pl.BlockSpec((d, half), lambda g: (0, g + nblk)),
        ],
        out_specs=pl.BlockSpec((half, 2 * d), lambda g: (g, 0)),
        out_shape=jax.ShapeDtypeStruct((nblk * half, 2 * d), jnp.float32),
    )(table_t, table_t)


def _sc_gather_sum(table_h, hidx3):
    nw, nchunk, ch = hidx3.shape
    bpw = nchunk // 4
    vh, dh = table_h.shape

    nbuf = 4
    unroll = 2

    def body(table_hbm, hidx_hbm, out_hbm, hidx_v, buf_v, out_v, *sems):
        wid = lax.axis_index("s") * NC + lax.axis_index("c")
        pltpu.sync_copy(hidx_hbm.at[wid], hidx_v)

        def fire(r, s):
            for c in range(4):
                pltpu.async_copy(
                    table_hbm.at[hidx_v.at[4 * r + c]],
                    buf_v.at[s, pl.ds(c * ch, ch)], sems[s])

        def drain(s):
            pltpu.make_async_copy(
                table_hbm.at[pl.ds(0, 4 * ch)], buf_v.at[s], sems[s]).wait()

        for s in range(nbuf):
            fire(s, s)

        zero = jnp.zeros((LANES,), jnp.float32)

        def group_body(g, carry):
            for s in range(nbuf):
                r = g * nbuf + s
                drain(s)

                def acc_body(i, accs):
                    a = list(accs)
                    for u in range(unroll):
                        row = i * unroll + u
                        for c in range(4):
                            a[c] = (a[c]
                                    + buf_v[s, (c // 2) * 2 * ch + row,
                                            pl.ds((c % 2) * LANES, LANES)]
                                    + buf_v[s, (c // 2) * 2 * ch + ch + row,
                                            pl.ds((c % 2) * LANES, LANES)])
                    return tuple(a)

                accs = lax.fori_loop(
                    0, ch // unroll, acc_body, (zero,) * 4)
                for c in range(4):
                    out_v[r, pl.ds(c * LANES, LANES)] = accs[c]

                @pl.when(r + nbuf < bpw)
                def _():
                    fire(r + nbuf, s)
            return carry

        lax.fori_loop(0, bpw // nbuf, group_body, 0)
        pltpu.sync_copy(out_v, out_hbm.at[pl.ds(wid * bpw, bpw)])

    run = pl.kernel(
        body,
        out_type=jax.ShapeDtypeStruct((nw * bpw, 2 * dh), jnp.float32),
        mesh=plsc.VectorSubcoreMesh(core_axis_name="c", subcore_axis_name="s"),
        scratch_types=[
            pltpu.VMEM((nchunk, ch), jnp.int32),
            pltpu.VMEM((nbuf, 4 * ch, dh), jnp.float32),
            pltpu.VMEM((bpw, 2 * dh), jnp.float32),
        ] + [pltpu.SemaphoreType.DMA] * nbuf,
        compiler_params=pltpu.CompilerParams(use_tc_tiling_on_sc=False),
    )
    return run(table_h, hidx3)


def _mlp(sums, w1, b1, w2t, b2, inv_l):
    b, d = sums.shape

    def body(s_ref, w1_ref, b1_ref, w2_ref, b2_ref, o_ref):
        feats = s_ref[...].astype(jnp.float32) * inv_l
        hid = jnp.dot(feats, w1_ref[...], preferred_element_type=jnp.float32)
        hid = jnp.maximum(hid + b1_ref[...], 0.0)
        logits = (jnp.sum(hid * w2_ref[...], axis=1, keepdims=True)
                  + b2_ref[...])
        mx = jnp.max(logits, axis=1, keepdims=True)
        e = jnp.exp(logits - mx)
        o_ref[...] = e / jnp.sum(e, axis=1, keepdims=True)

    return pl.pallas_call(
        body,
        out_shape=jax.ShapeDtypeStruct((b, 1), jnp.float32),
    )(sums, w1, b1, w2t, b2)


def kernel(inputs, table, W1, b1, W2, b2):
    b, l = inputs.shape
    v, d = table.shape
    cols = 1024
    half = cols // 2
    voff = pl.cdiv(v, cols) * half
    idx = inputs.astype(jnp.int32)
    left = idx < voff
    base = jnp.where(left, 4 * idx, 4 * (idx - voff) + 2)
    lo = base.reshape(b, 2, l // 2)
    hi = (base + 1).reshape(b, 2, l // 2)
    hidx3 = jnp.concatenate([lo, hi], axis=1)
    hidx3 = hidx3.reshape(NW, (b // NW) * 4, l // 2)
    table_h = _relayout_pack(table.T, cols).reshape(4 * voff, d // 2)
    sums = _sc_gather_sum(table_h, hidx3)
    return _mlp(sums, W1, b1.reshape(1, -1), W2.reshape(1, -1),
                b2.reshape(1, 1), 1.0 / l)


# speedup vs baseline: 1.7925x; 1.7925x over previous
import functools

import jax
import jax.numpy as jnp
from jax import lax
from jax.experimental import pallas as pl
from jax.experimental.pallas import tpu as pltpu
from jax.experimental.pallas import tpu_sc as plsc

NC = 2
NS = 16
NW = NC * NS
LANES = 16


def _relayout_pack(table_t, cols):
    d, v = table_t.shape
    half = cols // 2
    nblk = pl.cdiv(v, cols)
    last = pl.cdiv(v, half) - 1

    def body(tl, tr, tout):
        tout[:, 0:d] = tl[...].T
        tout[:, d:2 * d] = tr[...].T

    return pl.pallas_call(
        body,
        grid=(nblk,),
        in_specs=[
            pl.BlockSpec((d, half), lambda g: (0, g)),
            pl.BlockSpec((d, half), lambda g: (0, jnp.minimum(g + nblk, last))),
        ],
        out_specs=pl.BlockSpec((half, 2 * d), lambda g: (g, 0)),
        out_shape=jax.ShapeDtypeStruct((nblk * half, 2 * d), jnp.float32),
    )(table_t, table_t)


def _sc_gather_sum(table_h, hidx3):
    nw, nchunk, ch = hidx3.shape
    bpw = nchunk // 4
    vh, dh = table_h.shape

    nbuf = 4
    unroll = 2

    def body(table_hbm, hidx_hbm, out_hbm, hidx_v, buf_v, out_v, *sems):
        wid = lax.axis_index("s") * NC + lax.axis_index("c")
        pltpu.sync_copy(hidx_hbm.at[wid], hidx_v)

        def fire(r, s):
            for c in range(4):
                pltpu.async_copy(
                    table_hbm.at[hidx_v.at[4 * r + c]],
                    buf_v.at[s, pl.ds(c * ch, ch)], sems[s])

        def drain(s):
            pltpu.make_async_copy(
                table_hbm.at[pl.ds(0, 4 * ch)], buf_v.at[s], sems[s]).wait()

        for s in range(nbuf):
            fire(s, s)

        zero = jnp.zeros((LANES,), jnp.float32)

        def group_body(g, carry):
            for s in range(nbuf):
                r = g * nbuf + s
                drain(s)

                def acc_body(i, accs):
                    a = list(accs)
                    for u in range(unroll):
                        row = i * unroll + u
                        for c in range(4):
                            a[c] = (a[c]
                                    + buf_v[s, (c // 2) * 2 * ch + row,
                                            pl.ds((c % 2) * LANES, LANES)]
                                    + buf_v[s, (c // 2) * 2 * ch + ch + row,
                                            pl.ds((c % 2) * LANES, LANES)])
                    return tuple(a)

                accs = lax.fori_loop(
                    0, ch // unroll, acc_body, (zero,) * 4)
                for c in range(4):
                    out_v[r, pl.ds(c * LANES, LANES)] = accs[c]

                @pl.when(r + nbuf < bpw)
                def _():
                    fire(r + nbuf, s)
            return carry

        lax.fori_loop(0, bpw // nbuf, group_body, 0)
        pltpu.sync_copy(out_v, out_hbm.at[pl.ds(wid * bpw, bpw)])

    run = pl.kernel(
        body,
        out_type=jax.ShapeDtypeStruct((nw * bpw, 2 * dh), jnp.float32),
        mesh=plsc.VectorSubcoreMesh(core_axis_name="c", subcore_axis_name="s"),
        scratch_types=[
            pltpu.VMEM((nchunk, ch), jnp.int32),
            pltpu.VMEM((nbuf, 4 * ch, dh), jnp.float32),
            pltpu.VMEM((bpw, 2 * dh), jnp.float32),
        ] + [pltpu.SemaphoreType.DMA] * nbuf,
        compiler_params=pltpu.CompilerParams(use_tc_tiling_on_sc=False),
    )
    return run(table_h, hidx3)


def _mlp(sums, w1, b1, w2t, b2, inv_l):
    b, d = sums.shape

    def body(s_ref, w1_ref, b1_ref, w2_ref, b2_ref, o_ref):
        feats = s_ref[...].astype(jnp.float32) * inv_l
        hid = jnp.dot(feats, w1_ref[...], preferred_element_type=jnp.float32)
        hid = jnp.maximum(hid + b1_ref[...], 0.0)
        logits = (jnp.sum(hid * w2_ref[...], axis=1, keepdims=True)
                  + b2_ref[...])
        mx = jnp.max(logits, axis=1, keepdims=True)
        e = jnp.exp(logits - mx)
        o_ref[...] = e / jnp.sum(e, axis=1, keepdims=True)

    return pl.pallas_call(
        body,
        out_shape=jax.ShapeDtypeStruct((b, 1), jnp.float32),
    )(sums, w1, b1, w2t, b2)


def kernel(inputs, table, W1, b1, W2, b2):
    b, l = inputs.shape
    v, d = table.shape
    cols = 4096
    half = cols // 2
    voff = pl.cdiv(v, cols) * half
    idx = inputs.astype(jnp.int32)
    left = idx < voff
    base = jnp.where(left, 4 * idx, 4 * (idx - voff) + 2)
    lo = base.reshape(b, 2, l // 2)
    hi = (base + 1).reshape(b, 2, l // 2)
    hidx3 = jnp.concatenate([lo, hi], axis=1)
    hidx3 = hidx3.reshape(NW, (b // NW) * 4, l // 2)
    table_h = _relayout_pack(table.T, cols).reshape(4 * voff, d // 2)
    sums = _sc_gather_sum(table_h, hidx3)
    return _mlp(sums, W1, b1.reshape(1, -1), W2.reshape(1, -1),
                b2.reshape(1, 1), 1.0 / l)


# speedup vs baseline: 2.0849x; 1.1631x over previous
import functools

import jax
import jax.numpy as jnp
from jax import lax
from jax.experimental import pallas as pl
from jax.experimental.pallas import tpu as pltpu
from jax.experimental.pallas import tpu_sc as plsc

NC = 2
NS = 16
NW = NC * NS
LANES = 16


def _relayout_pack(table_t, cols):
    d, v = table_t.shape
    half = cols // 2
    nblk = pl.cdiv(v, cols)
    last = pl.cdiv(v, half) - 1

    def body(tl, tr, tout):
        tout[:, 0:d] = tl[...].T
        tout[:, d:2 * d] = tr[...].T

    return pl.pallas_call(
        body,
        grid=(nblk,),
        in_specs=[
            pl.BlockSpec((d, half), lambda g: (0, g)),
            pl.BlockSpec((d, half), lambda g: (0, jnp.minimum(g + nblk, last))),
        ],
        out_specs=pl.BlockSpec((half, 2 * d), lambda g: (g, 0)),
        out_shape=jax.ShapeDtypeStruct((nblk * half, 2 * d), jnp.float32),
    )(table_t, table_t)


def _sc_gather_sum(table_h, hidx3):
    nw, nchunk, ch = hidx3.shape
    bpw = nchunk // 4
    vh, dh = table_h.shape

    nbuf = 4
    unroll = 2

    def body(table_hbm, hidx_hbm, out_hbm, hidx_v, buf_v, out_v, *sems):
        wid = lax.axis_index("s") * NC + lax.axis_index("c")
        pltpu.sync_copy(hidx_hbm.at[wid], hidx_v)

        def fire(r, s):
            for c in range(4):
                pltpu.async_copy(
                    table_hbm.at[hidx_v.at[4 * r + c]],
                    buf_v.at[s, pl.ds(c * ch, ch)], sems[s])

        def drain(s):
            pltpu.make_async_copy(
                table_hbm.at[pl.ds(0, 4 * ch)], buf_v.at[s], sems[s]).wait()

        for s in range(nbuf):
            fire(s, s)

        zero = jnp.zeros((LANES,), jnp.float32)

        def group_body(g, carry):
            for s in range(nbuf):
                r = g * nbuf + s
                drain(s)

                def acc_body(i, accs):
                    a = list(accs)
                    for u in range(unroll):
                        row = i * unroll + u
                        for c in range(4):
                            a[c] = (a[c]
                                    + buf_v[s, (c // 2) * 2 * ch + row,
                                            pl.ds((c % 2) * LANES, LANES)]
                                    + buf_v[s, (c // 2) * 2 * ch + ch + row,
                                            pl.ds((c % 2) * LANES, LANES)])
                    return tuple(a)

                accs = lax.fori_loop(
                    0, ch // unroll, acc_body, (zero,) * 4)
                for c in range(4):
                    out_v[r, pl.ds(c * LANES, LANES)] = accs[c]

                @pl.when(r + nbuf < bpw)
                def _():
                    fire(r + nbuf, s)
            return carry

        lax.fori_loop(0, bpw // nbuf, group_body, 0)
        pltpu.sync_copy(out_v, out_hbm.at[pl.ds(wid * bpw, bpw)])

    run = pl.kernel(
        body,
        out_type=jax.ShapeDtypeStruct((nw * bpw, 2 * dh), jnp.float32),
        mesh=plsc.VectorSubcoreMesh(core_axis_name="c", subcore_axis_name="s"),
        scratch_types=[
            pltpu.VMEM((nchunk, ch), jnp.int32),
            pltpu.VMEM((nbuf, 4 * ch, dh), jnp.float32),
            pltpu.VMEM((bpw, 2 * dh), jnp.float32),
        ] + [pltpu.SemaphoreType.DMA] * nbuf,
        compiler_params=pltpu.CompilerParams(use_tc_tiling_on_sc=False),
    )
    return run(table_h, hidx3)


def _mlp(sums, w1, b1, w2t, b2, inv_l):
    b, d = sums.shape

    def body(s_ref, w1_ref, b1_ref, w2_ref, b2_ref, o_ref):
        feats = s_ref[...].astype(jnp.float32) * inv_l
        hid = jnp.dot(feats, w1_ref[...], preferred_element_type=jnp.float32)
        hid = jnp.maximum(hid + b1_ref[...], 0.0)
        logits = (jnp.sum(hid * w2_ref[...], axis=1, keepdims=True)
                  + b2_ref[...])
        mx = jnp.max(logits, axis=1, keepdims=True)
        e = jnp.exp(logits - mx)
        o_ref[...] = e / jnp.sum(e, axis=1, keepdims=True)

    return pl.pallas_call(
        body,
        out_shape=jax.ShapeDtypeStruct((b, 1), jnp.float32),
    )(sums, w1, b1, w2t, b2)


def kernel(inputs, table, W1, b1, W2, b2):
    b, l = inputs.shape
    v, d = table.shape
    cols = 8192
    half = cols // 2
    voff = pl.cdiv(v, cols) * half
    idx = inputs.astype(jnp.int32)
    left = idx < voff
    base = jnp.where(left, 4 * idx, 4 * (idx - voff) + 2)
    lo = base.reshape(b, 2, l // 2)
    hi = (base + 1).reshape(b, 2, l // 2)
    hidx3 = jnp.concatenate([lo, hi], axis=1)
    hidx3 = hidx3.reshape(NW, (b // NW) * 4, l // 2)
    table_h = _relayout_pack(table.T, cols).reshape(4 * voff, d // 2)
    sums = _sc_gather_sum(table_h, hidx3)
    return _mlp(sums, W1, b1.reshape(1, -1), W2.reshape(1, -1),
                b2.reshape(1, 1), 1.0 / l)


# speedup vs baseline: 2.2696x; 1.0886x over previous
import functools

import jax
import jax.numpy as jnp
from jax import lax
from jax.experimental import pallas as pl
from jax.experimental.pallas import tpu as pltpu
from jax.experimental.pallas import tpu_sc as plsc

NC = 2
NS = 16
NW = NC * NS
LANES = 16


def _relayout_pack(table_t, cols):
    d, v = table_t.shape
    half = cols // 2
    nblk = pl.cdiv(v, cols)
    last = pl.cdiv(v, half) - 1

    def body(tl, tr, tout):
        tout[:, 0:d] = tl[...].T
        tout[:, d:2 * d] = tr[...].T

    return pl.pallas_call(
        body,
        grid=(nblk,),
        in_specs=[
            pl.BlockSpec((d, half), lambda g: (0, g)),
            pl.BlockSpec((d, half), lambda g: (0, jnp.minimum(g + nblk, last))),
        ],
        out_specs=pl.BlockSpec((half, 2 * d), lambda g: (g, 0)),
        out_shape=jax.ShapeDtypeStruct((nblk * half, 2 * d), jnp.float32),
    )(table_t, table_t)


def _sc_gather_sum(table_h, hidx3):
    nw, nchunk, ch = hidx3.shape
    bpw = nchunk // 4
    vh, dh = table_h.shape

    nbuf = 4
    unroll = 2

    def body(table_hbm, hidx_hbm, out_hbm, hidx_v, buf_v, out_v, *sems):
        wid = lax.axis_index("s") * NC + lax.axis_index("c")
        pltpu.sync_copy(hidx_hbm.at[wid], hidx_v)

        def fire(r, s):
            for c in range(4):
                pltpu.async_copy(
                    table_hbm.at[hidx_v.at[4 * r + c]],
                    buf_v.at[s, pl.ds(c * ch, ch)], sems[s])

        def drain(s):
            pltpu.make_async_copy(
                table_hbm.at[pl.ds(0, 4 * ch)], buf_v.at[s], sems[s]).wait()

        for s in range(nbuf):
            fire(s, s)

        zero = jnp.zeros((LANES,), jnp.float32)

        def group_body(g, carry):
            for s in range(nbuf):
                r = g * nbuf + s
                drain(s)

                def acc_body(i, accs):
                    a = list(accs)
                    for u in range(unroll):
                        row = i * unroll + u
                        for c in range(4):
                            a[c] = (a[c]
                                    + buf_v[s, (c // 2) * 2 * ch + row,
                                            pl.ds((c % 2) * LANES, LANES)]
                                    + buf_v[s, (c // 2) * 2 * ch + ch + row,
                                            pl.ds((c % 2) * LANES, LANES)])
                    return tuple(a)

                accs = lax.fori_loop(
                    0, ch // unroll, acc_body, (zero,) * 4)
                for c in range(4):
                    out_v[r, pl.ds(c * LANES, LANES)] = accs[c]

                @pl.when(r + nbuf < bpw)
                def _():
                    fire(r + nbuf, s)
            return carry

        lax.fori_loop(0, bpw // nbuf, group_body, 0)
        pltpu.sync_copy(out_v, out_hbm.at[pl.ds(wid * bpw, bpw)])

    run = pl.kernel(
        body,
        out_type=jax.ShapeDtypeStruct((nw * bpw, 2 * dh), jnp.float32),
        mesh=plsc.VectorSubcoreMesh(core_axis_name="c", subcore_axis_name="s"),
        scratch_types=[
            pltpu.VMEM((nchunk, ch), jnp.int32),
            pltpu.VMEM((nbuf, 4 * ch, dh), jnp.float32),
            pltpu.VMEM((bpw, 2 * dh), jnp.float32),
        ] + [pltpu.SemaphoreType.DMA] * nbuf,
        compiler_params=pltpu.CompilerParams(use_tc_tiling_on_sc=False),
    )
    return run(table_h, hidx3)


def _mlp(sums, w1, b1, w2t, b2, inv_l):
    b, d = sums.shape

    def body(s_ref, w1_ref, b1_ref, w2_ref, b2_ref, o_ref):
        feats = s_ref[...].astype(jnp.float32) * inv_l
        hid = jnp.dot(feats, w1_ref[...], preferred_element_type=jnp.float32)
        hid = jnp.maximum(hid + b1_ref[...], 0.0)
        logits = (jnp.sum(hid * w2_ref[...], axis=1, keepdims=True)
                  + b2_ref[...])
        mx = jnp.max(logits, axis=1, keepdims=True)
        e = jnp.exp(logits - mx)
        o_ref[...] = e / jnp.sum(e, axis=1, keepdims=True)

    return pl.pallas_call(
        body,
        out_shape=jax.ShapeDtypeStruct((b, 1), jnp.float32),
    )(sums, w1, b1, w2t, b2)


def kernel(inputs, table, W1, b1, W2, b2):
    b, l = inputs.shape
    v, d = table.shape
    cols = 16384
    half = cols // 2
    voff = pl.cdiv(v, cols) * half
    idx = inputs.astype(jnp.int32)
    left = idx < voff
    base = jnp.where(left, 4 * idx, 4 * (idx - voff) + 2)
    lo = base.reshape(b, 2, l // 2)
    hi = (base + 1).reshape(b, 2, l // 2)
    hidx3 = jnp.concatenate([lo, hi], axis=1)
    hidx3 = hidx3.reshape(NW, (b // NW) * 4, l // 2)
    table_h = _relayout_pack(table.T, cols).reshape(4 * voff, d // 2)
    sums = _sc_gather_sum(table_h, hidx3)
    return _mlp(sums, W1, b1.reshape(1, -1), W2.reshape(1, -1),
                b2.reshape(1, 1), 1.0 / l)


# speedup vs baseline: 2.3635x; 1.0414x over previous
import functools

import jax
import jax.numpy as jnp
from jax import lax
from jax.experimental import pallas as pl
from jax.experimental.pallas import tpu as pltpu
from jax.experimental.pallas import tpu_sc as plsc

NC = 2
NS = 16
NW = NC * NS
LANES = 16


def _relayout_pack(table_t, cols):
    d, v = table_t.shape
    half = cols // 2
    nblk = pl.cdiv(v, cols)
    last = pl.cdiv(v, half) - 1

    def body(tl, tr, tout):
        tout[:, 0:d] = tl[...].T
        tout[:, d:2 * d] = tr[...].T

    return pl.pallas_call(
        body,
        grid=(nblk,),
        in_specs=[
            pl.BlockSpec((d, half), lambda g: (0, g)),
            pl.BlockSpec((d, half), lambda g: (0, jnp.minimum(g + nblk, last))),
        ],
        out_specs=pl.BlockSpec((half, 2 * d), lambda g: (g, 0)),
        out_shape=jax.ShapeDtypeStruct((nblk * half, 2 * d), jnp.float32),
    )(table_t, table_t)


def _sc_gather_sum(table_h, hidx3):
    nw, nchunk, ch = hidx3.shape
    bpw = nchunk // 4
    vh, dh = table_h.shape

    nbuf = 4
    unroll = 4

    def body(table_hbm, hidx_hbm, out_hbm, hidx_v, buf_v, out_v, *sems):
        wid = lax.axis_index("s") * NC + lax.axis_index("c")
        pltpu.sync_copy(hidx_hbm.at[wid], hidx_v)

        def fire(r, s):
            for c in range(4):
                pltpu.async_copy(
                    table_hbm.at[hidx_v.at[4 * r + c]],
                    buf_v.at[s, pl.ds(c * ch, ch)], sems[s])

        def drain(s):
            pltpu.make_async_copy(
                table_hbm.at[pl.ds(0, 4 * ch)], buf_v.at[s], sems[s]).wait()

        for s in range(nbuf):
            fire(s, s)

        zero = jnp.zeros((LANES,), jnp.float32)

        def group_body(g, carry):
            for s in range(nbuf):
                r = g * nbuf + s
                drain(s)

                def acc_body(i, accs):
                    a = list(accs)
                    for u in range(unroll):
                        row = i * unroll + u
                        for c in range(4):
                            a[c] = (a[c]
                                    + buf_v[s, (c // 2) * 2 * ch + row,
                                            pl.ds((c % 2) * LANES, LANES)]
                                    + buf_v[s, (c // 2) * 2 * ch + ch + row,
                                            pl.ds((c % 2) * LANES, LANES)])
                    return tuple(a)

                accs = lax.fori_loop(
                    0, ch // unroll, acc_body, (zero,) * 4)
                for c in range(4):
                    out_v[r, pl.ds(c * LANES, LANES)] = accs[c]

                @pl.when(r + nbuf < bpw)
                def _():
                    fire(r + nbuf, s)
            return carry

        lax.fori_loop(0, bpw // nbuf, group_body, 0)
        pltpu.sync_copy(out_v, out_hbm.at[pl.ds(wid * bpw, bpw)])

    run = pl.kernel(
        body,
        out_type=jax.ShapeDtypeStruct((nw * bpw, 2 * dh), jnp.float32),
        mesh=plsc.VectorSubcoreMesh(core_axis_name="c", subcore_axis_name="s"),
        scratch_types=[
            pltpu.VMEM((nchunk, ch), jnp.int32),
            pltpu.VMEM((nbuf, 4 * ch, dh), jnp.float32),
            pltpu.VMEM((bpw, 2 * dh), jnp.float32),
        ] + [pltpu.SemaphoreType.DMA] * nbuf,
        compiler_params=pltpu.CompilerParams(use_tc_tiling_on_sc=False),
    )
    return run(table_h, hidx3)


def _mlp(sums, w1, b1, w2t, b2, inv_l):
    b, d = sums.shape

    def body(s_ref, w1_ref, b1_ref, w2_ref, b2_ref, o_ref):
        feats = s_ref[...].astype(jnp.float32) * inv_l
        hid = jnp.dot(feats, w1_ref[...], preferred_element_type=jnp.float32)
        hid = jnp.maximum(hid + b1_ref[...], 0.0)
        logits = (jnp.sum(hid * w2_ref[...], axis=1, keepdims=True)
                  + b2_ref[...])
        mx = jnp.max(logits, axis=1, keepdims=True)
        e = jnp.exp(logits - mx)
        o_ref[...] = e / jnp.sum(e, axis=1, keepdims=True)

    return pl.pallas_call(
        body,
        out_shape=jax.ShapeDtypeStruct((b, 1), jnp.float32),
    )(sums, w1, b1, w2t, b2)


def kernel(inputs, table, W1, b1, W2, b2):
    b, l = inputs.shape
    v, d = table.shape
    cols = 24576
    half = cols // 2
    voff = pl.cdiv(v, cols) * half
    idx = inputs.astype(jnp.int32)
    left = idx < voff
    base = jnp.where(left, 4 * idx, 4 * (idx - voff) + 2)
    lo = base.reshape(b, 2, l // 2)
    hi = (base + 1).reshape(b, 2, l // 2)
    hidx3 = jnp.concatenate([lo, hi], axis=1)
    hidx3 = hidx3.reshape(NW, (b // NW) * 4, l // 2)
    table_h = _relayout_pack(table.T, cols).reshape(4 * voff, d // 2)
    sums = _sc_gather_sum(table_h, hidx3)
    return _mlp(sums, W1, b1.reshape(1, -1), W2.reshape(1, -1),
                b2.reshape(1, 1), 1.0 / l)


# speedup vs baseline: 2.3690x; 1.0023x over previous
import functools

import jax
import jax.numpy as jnp
from jax import lax
from jax.experimental import pallas as pl
from jax.experimental.pallas import tpu as pltpu
from jax.experimental.pallas import tpu_sc as plsc

NC = 2
NS = 16
NW = NC * NS
LANES = 16


def _relayout_pack(table_t, cols):
    d, v = table_t.shape
    half = cols // 2
    nblk = pl.cdiv(v, cols)
    last = pl.cdiv(v, half) - 1

    def body(tl, tr, tout):
        tout[:, 0:d] = tl[...].T
        tout[:, d:2 * d] = tr[...].T

    return pl.pallas_call(
        body,
        grid=(nblk,),
        in_specs=[
            pl.BlockSpec((d, half), lambda g: (0, g)),
            pl.BlockSpec((d, half), lambda g: (0, jnp.minimum(g + nblk, last))),
        ],
        out_specs=pl.BlockSpec((half, 2 * d), lambda g: (g, 0)),
        out_shape=jax.ShapeDtypeStruct((nblk * half, 2 * d), jnp.float32),
    )(table_t, table_t)


def _sc_gather_sum(table_h, hidx3):
    nw, nchunk, ch = hidx3.shape
    bpw = nchunk // 4
    vh, dh = table_h.shape

    nbuf = 4
    unroll = 4

    def body(table_hbm, hidx_hbm, out_hbm, hidx_v, buf_v, out_v, *sems):
        wid = lax.axis_index("s") * NC + lax.axis_index("c")
        pltpu.sync_copy(hidx_hbm.at[wid], hidx_v)

        def fire(r, s):
            for c in range(4):
                pltpu.async_copy(
                    table_hbm.at[hidx_v.at[4 * r + c]],
                    buf_v.at[s, pl.ds(c * ch, ch)], sems[s])

        def drain(s):
            pltpu.make_async_copy(
                table_hbm.at[pl.ds(0, 4 * ch)], buf_v.at[s], sems[s]).wait()

        for s in range(nbuf):
            fire(s, s)

        zero = jnp.zeros((LANES,), jnp.float32)

        def group_body(g, carry):
            for s in range(nbuf):
                r = g * nbuf + s
                drain(s)

                def acc_body(i, accs):
                    a = list(accs)
                    for u in range(unroll):
                        row = i * unroll + u
                        for c in range(4):
                            a[c] = (a[c]
                                    + buf_v[s, (c // 2) * 2 * ch + row,
                                            pl.ds((c % 2) * LANES, LANES)]
                                    + buf_v[s, (c // 2) * 2 * ch + ch + row,
                                            pl.ds((c % 2) * LANES, LANES)])
                    return tuple(a)

                accs = lax.fori_loop(
                    0, ch // unroll, acc_body, (zero,) * 4)
                for c in range(4):
                    out_v[r, pl.ds(c * LANES, LANES)] = accs[c]

                @pl.when(r + nbuf < bpw)
                def _():
                    fire(r + nbuf, s)
            return carry

        lax.fori_loop(0, bpw // nbuf, group_body, 0)
        pltpu.sync_copy(out_v, out_hbm.at[pl.ds(wid * bpw, bpw)])

    run = pl.kernel(
        body,
        out_type=jax.ShapeDtypeStruct((nw * bpw, 2 * dh), jnp.float32),
        mesh=plsc.VectorSubcoreMesh(core_axis_name="c", subcore_axis_name="s"),
        scratch_types=[
            pltpu.VMEM((nchunk, ch), jnp.int32),
            pltpu.VMEM((nbuf, 4 * ch, dh), jnp.float32),
            pltpu.VMEM((bpw, 2 * dh), jnp.float32),
        ] + [pltpu.SemaphoreType.DMA] * nbuf,
        compiler_params=pltpu.CompilerParams(use_tc_tiling_on_sc=False),
    )
    return run(table_h, hidx3)


def _mlp(sums, w1, b1, w2t, b2, inv_l):
    b, d = sums.shape

    def body(s_ref, w1_ref, b1_ref, w2_ref, b2_ref, o_ref):
        feats = s_ref[...].astype(jnp.float32) * inv_l
        hid = jnp.dot(feats, w1_ref[...], preferred_element_type=jnp.float32)
        hid = jnp.maximum(hid + b1_ref[...], 0.0)
        logits = (jnp.sum(hid * w2_ref[...], axis=1, keepdims=True)
                  + b2_ref[...])
        mx = jnp.max(logits, axis=1, keepdims=True)
        e = jnp.exp(logits - mx)
        o_ref[...] = e / jnp.sum(e, axis=1, keepdims=True)

    return pl.pallas_call(
        body,
        out_shape=jax.ShapeDtypeStruct((b, 1), jnp.float32),
    )(sums, w1, b1, w2t, b2)


def kernel(inputs, table, W1, b1, W2, b2):
    b, l = inputs.shape
    v, d = table.shape
    cols = 32768
    half = cols // 2
    voff = pl.cdiv(v, cols) * half
    idx = inputs.astype(jnp.int32)
    left = idx < voff
    base = jnp.where(left, 4 * idx, 4 * (idx - voff) + 2)
    lo = base.reshape(b, 2, l // 2)
    hi = (base + 1).reshape(b, 2, l // 2)
    hidx3 = jnp.concatenate([lo, hi], axis=1)
    hidx3 = hidx3.reshape(NW, (b // NW) * 4, l // 2)
    table_h = _relayout_pack(table.T, cols).reshape(4 * voff, d // 2)
    sums = _sc_gather_sum(table_h, hidx3)
    return _mlp(sums, W1, b1.reshape(1, -1), W2.reshape(1, -1),
                b2.reshape(1, 1), 1.0 / l)


# speedup vs baseline: 2.3714x; 1.0010x over previous
import jax
import jax.numpy as jnp
from jax import lax
from jax.experimental import pallas as pl
from jax.experimental.pallas import tpu as pltpu
from jax.experimental.pallas import tpu_sc as plsc

NC = 2
NS = 16
NW = NC * NS
LANES = 16


def _relayout_pack(table_t, cols):
    d, v = table_t.shape
    half = cols // 2
    nblk = pl.cdiv(v, cols)
    last = pl.cdiv(v, half) - 1

    def body(tl, tr, tout):
        tout[:, 0:d] = tl[...].T
        tout[:, d:2 * d] = tr[...].T

    return pl.pallas_call(
        body,
        grid=(nblk,),
        in_specs=[
            pl.BlockSpec((d, half), lambda g: (0, g)),
            pl.BlockSpec((d, half), lambda g: (0, jnp.minimum(g + nblk, last))),
        ],
        out_specs=pl.BlockSpec((half, 2 * d), lambda g: (g, 0)),
        out_shape=jax.ShapeDtypeStruct((nblk * half, 2 * d), jnp.float32),
    )(table_t, table_t)


def _sc_gather_sum(table_h, hidx3):
    nw, nchunk, ch = hidx3.shape
    bpw = nchunk // 4
    vh, dh = table_h.shape

    nbuf = 4
    unroll = 4

    def body(table_hbm, hidx_hbm, out_hbm, hidx_v, buf_v, out_v, *sems):
        wid = lax.axis_index("s") * NC + lax.axis_index("c")
        pltpu.sync_copy(hidx_hbm.at[wid], hidx_v)

        def fire(r, s):
            for c in range(4):
                pltpu.async_copy(
                    table_hbm.at[hidx_v.at[4 * r + c]],
                    buf_v.at[s, pl.ds(c * ch, ch)], sems[s])

        def drain(s):
            pltpu.make_async_copy(
                table_hbm.at[pl.ds(0, 4 * ch)], buf_v.at[s], sems[s]).wait()

        for s in range(nbuf):
            fire(s, s)

        zero = jnp.zeros((LANES,), jnp.float32)

        def group_body(g, carry):
            for s in range(nbuf):
                r = g * nbuf + s
                drain(s)

                def acc_body(i, accs):
                    a = list(accs)
                    for u in range(unroll):
                        row = i * unroll + u
                        for c in range(4):
                            a[c] = (a[c]
                                    + buf_v[s, (c // 2) * 2 * ch + row,
                                            pl.ds((c % 2) * LANES, LANES)]
                                    + buf_v[s, (c // 2) * 2 * ch + ch + row,
                                            pl.ds((c % 2) * LANES, LANES)])
                    return tuple(a)

                accs = lax.fori_loop(
                    0, ch // unroll, acc_body, (zero,) * 4)
                for c in range(4):
                    out_v[r, pl.ds(c * LANES, LANES)] = accs[c]

                @pl.when(r + nbuf < bpw)
                def _():
                    fire(r + nbuf, s)
            return carry

        lax.fori_loop(0, bpw // nbuf, group_body, 0)
        pltpu.sync_copy(out_v, out_hbm.at[pl.ds(wid * bpw, bpw)])

    run = pl.kernel(
        body,
        out_type=jax.ShapeDtypeStruct((nw * bpw, 2 * dh), jnp.float32),
        mesh=plsc.VectorSubcoreMesh(core_axis_name="c", subcore_axis_name="s"),
        scratch_types=[
            pltpu.VMEM((nchunk, ch), jnp.int32),
            pltpu.VMEM((nbuf, 4 * ch, dh), jnp.float32),
            pltpu.VMEM((bpw, 2 * dh), jnp.float32),
        ] + [pltpu.SemaphoreType.DMA] * nbuf,
        compiler_params=pltpu.CompilerParams(use_tc_tiling_on_sc=False),
    )
    return run(table_h, hidx3)


def _mlp(sums, w1, b1, w2t, b2, inv_l):
    b, d = sums.shape

    def body(s_ref, w1_ref, b1_ref, w2_ref, b2_ref, o_ref):
        feats = s_ref[...].astype(jnp.float32) * inv_l
        hid = jnp.dot(feats, w1_ref[...], preferred_element_type=jnp.float32)
        hid = jnp.maximum(hid + b1_ref[...], 0.0)
        logits = (jnp.sum(hid * w2_ref[...], axis=1, keepdims=True)
                  + b2_ref[...])
        mx = jnp.max(logits, axis=1, keepdims=True)
        e = jnp.exp(logits - mx)
        o_ref[...] = e / jnp.sum(e, axis=1, keepdims=True)

    return pl.pallas_call(
        body,
        out_shape=jax.ShapeDtypeStruct((b, 1), jnp.float32),
    )(sums, w1, b1, w2t, b2)


def kernel(inputs, table, W1, b1, W2, b2):
    b, l = inputs.shape
    v, d = table.shape
    cols = 32768
    half = cols // 2
    voff = pl.cdiv(v, cols) * half
    idx = inputs.astype(jnp.int32)
    left = idx < voff
    base = jnp.where(left, 4 * idx, 4 * (idx - voff) + 2)
    lo = base.reshape(b, 2, l // 2)
    hi = (base + 1).reshape(b, 2, l // 2)
    hidx3 = jnp.concatenate([lo, hi], axis=1)
    hidx3 = hidx3.reshape(NW, (b // NW) * 4, l // 2)
    table_h = _relayout_pack(table.T, cols).reshape(4 * voff, d // 2)
    sums = _sc_gather_sum(table_h, hidx3)
    return _mlp(sums, W1, b1.reshape(1, -1), W2.reshape(1, -1),
                b2.reshape(1, 1), 1.0 / l)
